# Initial kernel scaffold; baseline (speedup 1.0000x reference)
import jax
import jax.numpy as jnp
from jax.experimental import pallas as pl


def kernel(h, exited_so_far, W, b):
    raise NotImplementedError("write your pallas kernel here")


# speedup vs baseline: 1.4210x; 1.4210x over previous
import jax
import jax.numpy as jnp
from jax.experimental import pallas as pl
from jax.experimental.pallas import tpu as pltpu

_D_MODEL = 4096
_THRESHOLD = 0.5
_CAPACITY_FRACTION = 0.5
_ROWS = 512


def _matvec_body(h_ref, w_ref, b_ref, s_ref):
    logits = jnp.dot(h_ref[...], w_ref[...], preferred_element_type=jnp.float32)
    s_ref[...] = jax.nn.sigmoid(logits + b_ref[0, 0])


def _mask_body(k_cap, s_ref, e_ref, m_ref):
    s = s_ref[...]
    bT = s.shape
    keys = jax.lax.bitcast_convert_type(s, jnp.int32)

    def tau_body(i, tau):
        bit = jnp.int32(1) << (jnp.int32(30) - i)
        cand = tau | bit
        cnt = jnp.sum((keys >= cand).astype(jnp.int32), axis=1, keepdims=True)
        return jnp.where(cnt >= k_cap, cand, tau)

    tau0 = jnp.zeros((bT[0], 1), jnp.int32)
    tau = jax.lax.fori_loop(0, 31, tau_body, tau0)

    n_gt = jnp.sum((keys > tau).astype(jnp.int32), axis=1, keepdims=True)
    need = k_cap - n_gt
    tie = keys == tau
    idx = jax.lax.broadcasted_iota(jnp.int32, bT, 1)

    def m_body(i, m):
        bit = jnp.int32(1) << (jnp.int32(12) - i)
        cand = m | bit
        cnt = jnp.sum((tie & (idx < cand)).astype(jnp.int32), axis=1, keepdims=True)
        return jnp.where(cnt <= need, cand, m)

    m0 = jnp.zeros((bT[0], 1), jnp.int32)
    m_sel = jax.lax.fori_loop(0, 13, m_body, m0)

    in_topk = (keys > tau) | (tie & (idx < m_sel))
    out = in_topk & (s > _THRESHOLD) & (e_ref[...] == 0)
    m_ref[...] = out.astype(jnp.int32)


def kernel(h, exited_so_far, W, b):
    B, T, D = h.shape
    k_cap = max(1, min(T, int(_CAPACITY_FRACTION * T + 0.5)))

    h_flat = h.reshape(B * T, D)
    w_col = W.reshape(D, 1)
    b2 = b.reshape(1, 1)

    n_chunks = (B * T) // _ROWS
    scores_flat = pl.pallas_call(
        _matvec_body,
        grid=(n_chunks,),
        in_specs=[
            pl.BlockSpec((_ROWS, D), lambda i: (i, 0)),
            pl.BlockSpec((D, 1), lambda i: (0, 0)),
            pl.BlockSpec(memory_space=pltpu.SMEM),
        ],
        out_specs=pl.BlockSpec((_ROWS, 1), lambda i: (i, 0)),
        out_shape=jax.ShapeDtypeStruct((B * T, 1), jnp.float32),
    )(h_flat, w_col, b2)

    scores2d = scores_flat.reshape(B, T)
    exited2d = exited_so_far.reshape(B, T).astype(jnp.int32)

    mask2d = pl.pallas_call(
        lambda s_ref, e_ref, m_ref: _mask_body(k_cap, s_ref, e_ref, m_ref),
        in_specs=[
            pl.BlockSpec((B, T), lambda: (0, 0)),
            pl.BlockSpec((B, T), lambda: (0, 0)),
        ],
        out_specs=pl.BlockSpec((B, T), lambda: (0, 0)),
        out_shape=jax.ShapeDtypeStruct((B, T), jnp.int32),
    )(scores2d, exited2d)

    scores = scores_flat.reshape(B, T, 1)
    exit_mask = mask2d.astype(jnp.bool_).reshape(B, T, 1)
    return (scores, exit_mask)
